# baseline (speedup 1.0000x reference)
import jax
import jax.numpy as jnp
from jax.experimental import pallas as pl
from jax.experimental.pallas import tpu as pltpu


CBLK = 128


def _mp_kernel(seq_ref, adj_ref, seqc_ref, w_ref, ws_ref, b_ref, out_ref, h_ref):
    c_i = pl.program_id(1)

    @pl.when(c_i == 0)
    def _compute_h():
        h_ref[...] = jnp.dot(
            seq_ref[0], w_ref[...], preferred_element_type=jnp.float32
        )

    a_blk = (adj_ref[0] != 0).astype(jnp.float32)
    agg = jax.lax.dot_general(
        a_blk, h_ref[...], (((0,), (0,)), ((), ())),
        preferred_element_type=jnp.float32,
    )
    self_term = jnp.dot(
        seqc_ref[0], ws_ref[...], preferred_element_type=jnp.float32
    )
    out_ref[0] = agg + self_term + b_ref[...]


def kernel(seq, mask, adj_matrix, W, W_self, b):
    B, L, d = seq.shape
    del mask
    b2d = b.reshape(1, d)
    out = pl.pallas_call(
        _mp_kernel,
        grid=(B, L // CBLK),
        in_specs=[
            pl.BlockSpec((1, L, d), lambda i, j: (i, 0, 0)),
            pl.BlockSpec((1, L, CBLK), lambda i, j: (i, 0, j)),
            pl.BlockSpec((1, CBLK, d), lambda i, j: (i, j, 0)),
            pl.BlockSpec((d, d), lambda i, j: (0, 0)),
            pl.BlockSpec((d, d), lambda i, j: (0, 0)),
            pl.BlockSpec((1, d), lambda i, j: (0, 0)),
        ],
        out_specs=pl.BlockSpec((1, CBLK, d), lambda i, j: (i, j, 0)),
        out_shape=jax.ShapeDtypeStruct((B, L, d), jnp.float32),
        scratch_shapes=[pltpu.VMEM((L, d), jnp.float32)],
    )(seq, adj_matrix, seq, W, W_self, b2d)
    return out


# speedup vs baseline: 2.8559x; 2.8559x over previous
import jax
import jax.numpy as jnp
from jax.experimental import pallas as pl


GB = 2


def _mp_kernel(seq_ref, adj_ref, w_ref, ws_ref, b_ref, out_ref):
    for g in range(GB):
        x = seq_ref[g]
        a = (adj_ref[g] != 0).astype(jnp.float32)
        h = jnp.dot(x, w_ref[...], preferred_element_type=jnp.float32)
        agg = jax.lax.dot_general(
            a, h, (((0,), (0,)), ((), ())), preferred_element_type=jnp.float32
        )
        self_term = jnp.dot(x, ws_ref[...], preferred_element_type=jnp.float32)
        out_ref[g] = agg + self_term + b_ref[...]


def kernel(seq, mask, adj_matrix, W, W_self, b):
    B, L, d = seq.shape
    del mask
    b2d = b.reshape(1, d)
    out = pl.pallas_call(
        _mp_kernel,
        grid=(B // GB,),
        in_specs=[
            pl.BlockSpec((GB, L, d), lambda i: (i, 0, 0)),
            pl.BlockSpec((GB, L, L), lambda i: (i, 0, 0)),
            pl.BlockSpec((d, d), lambda i: (0, 0)),
            pl.BlockSpec((d, d), lambda i: (0, 0)),
            pl.BlockSpec((1, d), lambda i: (0, 0)),
        ],
        out_specs=pl.BlockSpec((GB, L, d), lambda i: (i, 0, 0)),
        out_shape=jax.ShapeDtypeStruct((B, L, d), jnp.float32),
    )(seq, adj_matrix, W, W_self, b2d)
    return out
